# baseline (speedup 1.0000x reference)
import functools
import math

import jax
import jax.numpy as jnp
from jax import lax
from jax.experimental import pallas as pl
from jax.experimental.pallas import tpu as pltpu
from jax.experimental.pallas import tpu_sc as plsc

_NC = 2
_NS = 16
_NW = _NC * _NS
_LANES = 16
_CHUNK = 512


def _emb_body(idx_hbm, tbl_hbm, out_hbm, idx_v, rows_v, sem,
              *, b_per_w, chunk, d, scale):
    nchunk = b_per_w // chunk
    wid = lax.axis_index("s") * _NC + lax.axis_index("c")
    base = wid * b_per_w
    pltpu.sync_copy(idx_hbm.at[pl.ds(base, b_per_w)], idx_v)

    def chunk_body(g, carry):
        off = pl.multiple_of(g * chunk, 8)
        pltpu.async_copy(tbl_hbm.at[idx_v.at[pl.ds(off, chunk)]],
                         rows_v, sem).wait()

        def row_body(r, c2):
            for dd in range(0, d, _LANES):
                rows_v[r, pl.ds(dd, _LANES)] = (
                    rows_v[r, pl.ds(dd, _LANES)] * scale)
            return c2

        lax.fori_loop(0, chunk, row_body, 0)
        pltpu.sync_copy(rows_v, out_hbm.at[pl.ds(base + off, chunk)])
        return carry

    lax.fori_loop(0, nchunk, chunk_body, 0)


def kernel(input, weight, sigma, length_scale):
    orig_shape = input.shape
    d = weight.shape[1]
    scale = math.sqrt(d)
    idx = input.reshape(-1).astype(jnp.int32)
    b = idx.shape[0]
    b_per_w = b // _NW
    chunk = _CHUNK

    mesh = plsc.VectorSubcoreMesh(core_axis_name="c", subcore_axis_name="s",
                                  num_cores=_NC, num_subcores=_NS)
    body = functools.partial(_emb_body, b_per_w=b_per_w, chunk=chunk,
                             d=d, scale=scale)
    out = pl.kernel(
        body,
        out_type=jax.ShapeDtypeStruct((b, d), jnp.float32),
        mesh=mesh,
        compiler_params=pltpu.CompilerParams(use_tc_tiling_on_sc=False),
        scratch_types=[
            pltpu.VMEM((b_per_w,), jnp.int32),
            pltpu.VMEM((chunk, d), jnp.float32),
            pltpu.SemaphoreType.DMA,
        ],
    )(idx, weight)
    return out.reshape(*orig_shape, d)


# speedup vs baseline: 1.0896x; 1.0896x over previous
import functools
import math

import jax
import jax.numpy as jnp
from jax import lax
from jax.experimental import pallas as pl
from jax.experimental.pallas import tpu as pltpu
from jax.experimental.pallas import tpu_sc as plsc

_NC = 2
_NS = 16
_NW = _NC * _NS
_LANES = 16
_CHUNK = 640
_NBUF = 4
_RUNROLL = 4


def _emb_body(idx_hbm, tbl_hbm, out_hbm, idx_v, rows_v, gsem,
              *, b_per_w, chunk, nbuf, d, scale):
    nchunk = b_per_w // chunk
    nouter = nchunk // nbuf
    wid = lax.axis_index("s") * _NC + lax.axis_index("c")
    base = wid * b_per_w
    pltpu.sync_copy(idx_hbm.at[pl.ds(base, b_per_w)], idx_v)

    def g_src(off):
        return tbl_hbm.at[idx_v.at[pl.ds(off, chunk)]]

    for b in range(nbuf):
        pltpu.async_copy(g_src(b * chunk), rows_v.at[b], gsem)

    def outer(p, carry):
        g0 = p * nbuf
        for b in range(nbuf):
            g = g0 + b
            off = pl.multiple_of(g * chunk, 8)
            pltpu.make_async_copy(g_src(off), rows_v.at[b], gsem).wait()

            def row_body(r4, c2):
                r = r4 * _RUNROLL
                for rr in range(_RUNROLL):
                    for dd in range(0, d, _LANES):
                        rows_v[b, r + rr, pl.ds(dd, _LANES)] = (
                            rows_v[b, r + rr, pl.ds(dd, _LANES)] * scale)
                return c2

            lax.fori_loop(0, chunk // _RUNROLL, row_body, 0)
            pltpu.sync_copy(rows_v.at[b], out_hbm.at[pl.ds(base + off, chunk)])

            nxt = g + nbuf

            @pl.when(nxt < nchunk)
            def _issue_next():
                noff = pl.multiple_of(nxt * chunk, 8)
                pltpu.async_copy(g_src(noff), rows_v.at[b], gsem)

        return carry

    lax.fori_loop(0, nouter, outer, 0)


def kernel(input, weight, sigma, length_scale):
    orig_shape = input.shape
    d = weight.shape[1]
    scale = math.sqrt(d)
    idx = input.reshape(-1).astype(jnp.int32)
    b = idx.shape[0]
    b_per_w = b // _NW

    mesh = plsc.VectorSubcoreMesh(core_axis_name="c", subcore_axis_name="s",
                                  num_cores=_NC, num_subcores=_NS)
    body = functools.partial(_emb_body, b_per_w=b_per_w, chunk=_CHUNK,
                             nbuf=_NBUF, d=d, scale=scale)
    out = pl.kernel(
        body,
        out_type=jax.ShapeDtypeStruct((b, d), jnp.float32),
        mesh=mesh,
        compiler_params=pltpu.CompilerParams(use_tc_tiling_on_sc=False),
        scratch_types=[
            pltpu.VMEM((b_per_w,), jnp.int32),
            pltpu.VMEM((_NBUF, _CHUNK, d), jnp.float32),
            pltpu.SemaphoreType.DMA,
        ],
    )(idx, weight)
    return out.reshape(*orig_shape, d)


# speedup vs baseline: 1.5311x; 1.4051x over previous
import functools
import math

import jax
import jax.numpy as jnp
from jax import lax
from jax.experimental import pallas as pl
from jax.experimental.pallas import tpu as pltpu
from jax.experimental.pallas import tpu_sc as plsc

_NC = 2
_NS = 16
_NW = _NC * _NS
_LANES = 16


def _emb_body(idxT_hbm, tbl_hbm, out_hbm, idxall, rows0, rows1, tile0,
              tile1, isem, gsem, ssem, *, nj, tpw, d, scale):
    wid = lax.axis_index("s") * _NC + lax.axis_index("c")
    colbase = wid * tpw
    tb0 = wid * (tpw // 128)
    ntb = tpw // 128
    rows = (rows0, rows1)
    tiles = (tile0, tile1)

    for j in range(nj):
        pltpu.async_copy(idxT_hbm.at[j, pl.ds(colbase, tpw)],
                         idxall.at[j], isem)
    for j in range(nj):
        pltpu.make_async_copy(idxT_hbm.at[j, pl.ds(colbase, tpw)],
                              idxall.at[j], isem).wait()

    def gather(j, b):
        return pltpu.async_copy(tbl_hbm.at[idxall.at[j]], rows[b], gsem)

    def stores(j, b, start):
        for tc in range(4):
            c = pltpu.make_async_copy(tiles[b].at[tc],
                                      out_hbm.at[j, tc, pl.ds(tb0, ntb)],
                                      ssem)
            if start:
                c.start()
            else:
                c.wait()

    gather(0, 0)
    iota = lax.iota(jnp.int32, _LANES)

    def pair(p, carry):
        for b in (0, 1):
            j = p * 2 + b

            @pl.when(j >= 2)
            def _drain():
                stores(j - 2, b, start=False)

            @pl.when(j + 1 < nj)
            def _next():
                gather(j + 1, 1 - b)

            pltpu.make_async_copy(tbl_hbm.at[idxall.at[j]], rows[b],
                                  gsem).wait()

            def cbody(c, c2, _b=b):
                tc = c // 8
                cm = c - tc * 8
                col = jnp.zeros((_LANES,), jnp.int32) + c
                for g in range(tpw // _LANES):
                    row = iota + (_LANES * g)
                    v = plsc.load_gather(rows[_b], [row, col]) * scale
                    tiles[_b][tc, g // 8, cm, pl.ds((g % 8) * _LANES,
                                                    _LANES)] = v
                return c2

            lax.fori_loop(0, d, cbody, 0)
            stores(j, b, start=True)
        return carry

    lax.fori_loop(0, nj // 2, pair, 0)
    stores(nj - 2, 0, start=False)
    stores(nj - 1, 1, start=False)


def kernel(input, weight, sigma, length_scale):
    n_tok, nj = input.shape
    d = weight.shape[1]
    scale = math.sqrt(d)
    tpw = n_tok // _NW
    idxT = jnp.swapaxes(input, 0, 1).astype(jnp.int32)

    mesh = plsc.VectorSubcoreMesh(core_axis_name="c", subcore_axis_name="s",
                                  num_cores=_NC, num_subcores=_NS)
    body = functools.partial(_emb_body, nj=nj, tpw=tpw, d=d, scale=scale)
    oT5 = pl.kernel(
        body,
        out_type=jax.ShapeDtypeStruct((nj, d // 8, n_tok // 128, 8, 128),
                                      jnp.float32),
        mesh=mesh,
        compiler_params=pltpu.CompilerParams(use_tc_tiling_on_sc=False,
                                             needs_layout_passes=False),
        scratch_types=[
            pltpu.VMEM((nj, tpw), jnp.int32),
            pltpu.VMEM((tpw, d), jnp.float32),
            pltpu.VMEM((tpw, d), jnp.float32),
            pltpu.VMEM((d // 8, tpw // 128, 8, 128), jnp.float32),
            pltpu.VMEM((d // 8, tpw // 128, 8, 128), jnp.float32),
            pltpu.SemaphoreType.DMA,
            pltpu.SemaphoreType.DMA,
            pltpu.SemaphoreType.DMA,
        ],
    )(idxT, weight)
    return oT5.transpose(2, 4, 0, 1, 3).reshape(n_tok, nj, d)


# speedup vs baseline: 2.7217x; 1.7776x over previous
import functools
import math

import jax
import jax.numpy as jnp
from jax import lax
from jax.experimental import pallas as pl
from jax.experimental.pallas import tpu as pltpu
from jax.experimental.pallas import tpu_sc as plsc

_NC = 2
_NS = 16
_NW = _NC * _NS
_LANES = 16
_PAD = 129


def _emb_body(idxT_hbm, tbl_hbm, out_hbm, idxall, rows0, rows1, tile0,
              tile1, isem, gsem, ssem, *, nj, tpw, d, scale):
    wid = lax.axis_index("s") * _NC + lax.axis_index("c")
    colbase = wid * tpw
    ntb = tpw // 128
    tb0 = wid * ntb
    rows = (rows0, rows1)
    tiles = (tile0, tile1)

    for j in range(nj):
        pltpu.async_copy(idxT_hbm.at[j, pl.ds(colbase, tpw)],
                         idxall.at[j], isem)
    for j in range(nj):
        pltpu.make_async_copy(idxT_hbm.at[j, pl.ds(colbase, tpw)],
                              idxall.at[j], isem).wait()

    def gather(j, b):
        return pltpu.async_copy(tbl_hbm.at[idxall.at[j]], rows[b], gsem)

    def stores(j, b, start):
        for tbl in range(ntb):
            for tc in range(d // 8):
                c = pltpu.make_async_copy(
                    tiles[b].at[tbl, tc, slice(None), pl.ds(0, 128)],
                    out_hbm.at[j, tc, tb0 + tbl], ssem)
                if start:
                    c.start()
                else:
                    c.wait()

    gather(0, 0)
    iota = lax.iota(jnp.int32, _LANES)
    tc_a = iota // 8
    cm_a = iota % 8
    tc_b = tc_a + 2
    zeros = jnp.zeros((_LANES,), jnp.int32)

    def pair(p, carry):
        for b in (0, 1):
            j = p * 2 + b

            @pl.when(j >= 2)
            def _drain():
                stores(j - 2, b, start=False)

            @pl.when(j + 1 < nj)
            def _next():
                gather(j + 1, 1 - b)

            pltpu.make_async_copy(tbl_hbm.at[idxall.at[j]], rows[b],
                                  gsem).wait()

            for tbl in range(ntb):
                tbase = tbl * 128
                tbl_v = zeros + tbl

                def tbody(k, c2, _b=b, _tbase=tbase, _tbl_v=tbl_v):
                    for u in range(4):
                        im = k * 4 + u
                        t = _tbase + im
                        va = rows[_b][t, pl.ds(0, _LANES)] * scale
                        vb = rows[_b][t, pl.ds(_LANES, _LANES)] * scale
                        im_v = zeros + im
                        plsc.store_scatter(tiles[_b],
                                           [_tbl_v, tc_a, cm_a, im_v], va)
                        plsc.store_scatter(tiles[_b],
                                           [_tbl_v, tc_b, cm_a, im_v], vb)
                    return c2

                lax.fori_loop(0, 32, tbody, 0)
            stores(j, b, start=True)
        return carry

    lax.fori_loop(0, nj // 2, pair, 0)
    stores(nj - 2, 0, start=False)
    stores(nj - 1, 1, start=False)


def kernel(input, weight, sigma, length_scale):
    n_tok, nj = input.shape
    d = weight.shape[1]
    scale = math.sqrt(d)
    tpw = n_tok // _NW
    idxT = jnp.swapaxes(input, 0, 1).astype(jnp.int32)

    mesh = plsc.VectorSubcoreMesh(core_axis_name="c", subcore_axis_name="s",
                                  num_cores=_NC, num_subcores=_NS)
    body = functools.partial(_emb_body, nj=nj, tpw=tpw, d=d, scale=scale)
    ntb = tpw // 128
    oT5 = pl.kernel(
        body,
        out_type=jax.ShapeDtypeStruct((nj, d // 8, n_tok // 128, 8, 128),
                                      jnp.float32),
        mesh=mesh,
        compiler_params=pltpu.CompilerParams(use_tc_tiling_on_sc=False,
                                             needs_layout_passes=False),
        scratch_types=[
            pltpu.VMEM((nj, tpw), jnp.int32),
            pltpu.VMEM((tpw, d), jnp.float32),
            pltpu.VMEM((tpw, d), jnp.float32),
            pltpu.VMEM((ntb, d // 8, 8, _PAD), jnp.float32),
            pltpu.VMEM((ntb, d // 8, 8, _PAD), jnp.float32),
            pltpu.SemaphoreType.DMA,
            pltpu.SemaphoreType.DMA,
            pltpu.SemaphoreType.DMA,
        ],
    )(idxT, weight)
    return oT5.transpose(2, 4, 0, 1, 3).reshape(n_tok, nj, d)
